# baseline (speedup 1.0000x reference)
import functools

import jax
import jax.numpy as jnp
from jax import lax
from jax.experimental import pallas as pl
from jax.experimental.pallas import tpu as pltpu
from jax.experimental.pallas import tpu_sc as plsc


def _lse_body(table_ref, lse_ref):
    t = table_ref[...]
    m = jnp.max(t, axis=1)
    s = jnp.sum(jnp.exp(t - m[:, None]), axis=1)
    lse_ref[...] = m + jnp.log(s)


def _compute_lse(table):
    V = table.shape[0]
    return pl.pallas_call(
        _lse_body,
        out_shape=jax.ShapeDtypeStruct((V,), jnp.float32),
    )(table)


@functools.lru_cache(maxsize=None)
def _make_sc_gather(N, V, C):
    NC, NS = 2, 16
    NW = NC * NS
    assert N % NW == 0 and N % 128 == 0
    BPW = N // NW
    NSTRIPE = (C + 7) // 8
    SPW = (NSTRIPE + NW - 1) // NW
    CPAD = NW * SPW * 8
    TPB = 4
    IB = TPB * 128
    NBATCH = N // IB
    assert NBATCH % 2 == 0
    GPL = BPW // 16
    NLG = (BPW + 127) // 128

    mesh = plsc.VectorSubcoreMesh(core_axis_name="c", subcore_axis_name="s")

    @functools.partial(
        pl.kernel,
        mesh=mesh,
        compiler_params=pltpu.CompilerParams(needs_layout_passes=False),
        out_type=(
            jax.ShapeDtypeStruct((NSTRIPE, 8, N), jnp.float32),
            jax.ShapeDtypeStruct((NW, 16), jnp.float32),
        ),
        scratch_types=[
            pltpu.VMEM((N,), jnp.int32),
            pltpu.VMEM((BPW,), jnp.int32),
            pltpu.VMEM((V,), jnp.float32),
            pltpu.VMEM((SPW * 8 * 8 * 128,), jnp.float32),
            pltpu.VMEM((2, SPW, 8, IB), jnp.float32),
            pltpu.VMEM((BPW,), jnp.int32),
            pltpu.VMEM((BPW,), jnp.float32),
            pltpu.VMEM((16,), jnp.float32),
            pltpu.SemaphoreType.DMA,
            pltpu.SemaphoreType.DMA,
        ],
    )
    def sc_kernel(tTflat_hbm, tflat_hbm, idx_hbm, tgt_hbm, lse_hbm,
                  out_hbm, part_hbm,
                  idx_v, tgt_v, lse_v, slab_v, abuf_v, fidx_v, tval_v, acc_v,
                  ssem, lsem):
        wid = lax.axis_index("s") * NC + lax.axis_index("c")
        base = wid * BPW
        s0 = wid * SPW
        pltpu.sync_copy(idx_hbm, idx_v)
        pltpu.sync_copy(tgt_hbm.at[pl.ds(base, BPW)], tgt_v)
        pltpu.sync_copy(lse_hbm, lse_v)
        pltpu.sync_copy(
            tTflat_hbm.at[pl.ds(s0 * 8 * 1024, SPW * 8 * 1024)], slab_v)

        @pl.loop(0, GPL)
        def _fidx(k):
            j0 = base + k * 16
            ivec = idx_v[pl.ds(j0, 16)]
            tvec = tgt_v[pl.ds(k * 16, 16)]
            fidx_v[pl.ds(k * 16, 16)] = ivec * C + tvec
        lgs = []
        for k in range(NLG):
            o = k * 128
            n = min(128, BPW - o)
            lgs.append(pltpu.async_copy(
                tflat_hbm.at[fidx_v.at[pl.ds(o, n)]],
                tval_v.at[pl.ds(o, n)], lsem))

        def _drain(b, st):
            sg = s0 + st

            @pl.when(sg < NSTRIPE)
            def _():
                pltpu.make_async_copy(
                    abuf_v.at[b, st],
                    out_hbm.at[s0 + st, :, pl.ds(0, IB)], ssem
                ).wait()

        def _assemble_and_fire(bb, b):
            i0 = bb * IB

            @pl.loop(0, TPB)
            def _tile(k):
                ivl = []
                for g in range(8):
                    ivl.append(idx_v[pl.ds(i0 + k * 128 + g * 16, 16)])
                for st in range(SPW):
                    for u in range(8):
                        rbase = (st * 8 + u) * 1024
                        for g, iv in enumerate(ivl):
                            val = plsc.load_gather(slab_v, [iv + rbase])
                            abuf_v[b, st, u,
                                   pl.ds(k * 128 + g * 16, 16)] = val
            for st in range(SPW):
                sg = s0 + st

                @pl.when(sg < NSTRIPE)
                def _():
                    pltpu.async_copy(
                        abuf_v.at[b, st],
                        out_hbm.at[s0 + st, :, pl.ds(i0, IB)], ssem)

        @pl.loop(0, NBATCH, step=2)
        def _batch(ci):
            for b in range(2):
                @pl.when(ci >= 2)
                def _():
                    for st in range(SPW):
                        _drain(b, st)
                _assemble_and_fire(ci + b, b)

        for b in range(2):
            for st in range(SPW):
                _drain(b, st)

        for cp in lgs:
            cp.wait()
        acc_v[...] = jnp.zeros((16,), jnp.float32)

        @pl.loop(0, GPL)
        def _loss(k):
            j0 = base + k * 16
            ivec = idx_v[pl.ds(j0, 16)]
            lsev = plsc.load_gather(lse_v, [ivec])
            acc_v[...] = acc_v[...] + (lsev - tval_v[pl.ds(k * 16, 16)])

        pltpu.sync_copy(acc_v, part_hbm.at[wid])

    return sc_kernel


def _make_fin(N):
    def _fin_body(part_ref, loss_ref):
        loss_ref[...] = (jnp.sum(part_ref[...]) / N).reshape(1, 1)

    return pl.pallas_call(
        _fin_body,
        out_shape=jax.ShapeDtypeStruct((1, 1), jnp.float32),
    )


def kernel(idx, targets, table):
    B, T = idx.shape
    V, C = table.shape
    N = B * T
    NSTRIPE = (C + 7) // 8
    CPAD = 1024 if C <= 1024 else ((C + 1023) // 1024) * 1024
    VPAD = (V + 127) // 128 * 128
    idx_flat = idx.reshape(N).astype(jnp.int32)
    tgt_flat = targets.reshape(N).astype(jnp.int32)
    table = table.astype(jnp.float32)
    tT_flat = jnp.pad(table.T, ((0, CPAD - C), (0, VPAD - V))).reshape(
        CPAD * VPAD)
    table_flat = table.reshape(V * C)

    lse = _compute_lse(table)
    outT3, partials = _make_sc_gather(N, V, C)(
        tT_flat, table_flat, idx_flat, tgt_flat, lse
    )
    logits = outT3.reshape(NSTRIPE * 8, N).T
    loss = _make_fin(N)(partials).reshape(())
    return (logits, loss)


# speedup vs baseline: 1.3418x; 1.3418x over previous
import functools

import jax
import jax.numpy as jnp
from jax import lax
from jax.experimental import pallas as pl
from jax.experimental.pallas import tpu as pltpu
from jax.experimental.pallas import tpu_sc as plsc


def _lse_body(table_ref, lse_ref):
    t = table_ref[...]
    m = jnp.max(t, axis=1)
    s = jnp.sum(jnp.exp(t - m[:, None]), axis=1)
    lse_ref[...] = m + jnp.log(s)


def _compute_lse(table):
    V = table.shape[0]
    return pl.pallas_call(
        _lse_body,
        out_shape=jax.ShapeDtypeStruct((V,), jnp.float32),
    )(table)


@functools.lru_cache(maxsize=None)
def _make_sc_gather(N, V, C):
    NC, NS = 2, 16
    NW = NC * NS
    assert N % NW == 0
    BPW = N // NW
    CH = 32
    assert BPW % CH == 0 and CH % 16 == 0 and NCHUNK_EVEN(BPW, CH)
    NCHUNK = BPW // CH
    G = CH // 16
    NT = (C + 127) // 128
    FULLT = C // 128
    TAILC = C - FULLT * 128
    FULLG = TAILC // 16
    REM = TAILC - FULLG * 16

    mesh = plsc.VectorSubcoreMesh(core_axis_name="c", subcore_axis_name="s")

    @functools.partial(
        pl.kernel,
        mesh=mesh,
        compiler_params=pltpu.CompilerParams(needs_layout_passes=False),
        out_type=(
            jax.ShapeDtypeStruct((N, C), jnp.float32),
            jax.ShapeDtypeStruct((NW, 16), jnp.float32),
        ),
        scratch_types=[
            pltpu.VMEM((BPW,), jnp.int32),
            pltpu.VMEM((BPW,), jnp.int32),
            pltpu.VMEM((V,), jnp.float32),
            pltpu.VMEM((CH, NT, 128), jnp.float32),
            pltpu.VMEM((CH, NT, 128), jnp.float32),
            pltpu.VMEM((CH, TAILC), jnp.float32),
            pltpu.VMEM((16,), jnp.float32),
            pltpu.SemaphoreType.DMA,
            pltpu.SemaphoreType.DMA,
        ],
    )
    def sc_kernel(t4_hbm, idx_hbm, tgt_hbm, lse_hbm, out_hbm, part_hbm,
                  idx_v, tgt_v, lse_v, rows_a, rows_b, tail_v, acc_v,
                  gsem, ssem):
        wid = lax.axis_index("s") * NC + lax.axis_index("c")
        base = wid * BPW
        pltpu.sync_copy(idx_hbm.at[pl.ds(base, BPW)], idx_v)
        pltpu.sync_copy(tgt_hbm.at[pl.ds(base, BPW)], tgt_v)
        pltpu.sync_copy(lse_hbm, lse_v)
        acc_v[...] = jnp.zeros((16,), jnp.float32)
        lanes = lax.broadcasted_iota(jnp.int32, (16,), 0)

        pltpu.async_copy(t4_hbm.at[idx_v.at[pl.ds(0, CH)]], rows_a, gsem)

        @pl.loop(0, NCHUNK, step=2)
        def _outer(ci):
            for b, (buf, obuf) in enumerate(((rows_a, rows_b),
                                             (rows_b, rows_a))):
                c = ci + b
                off = c * CH
                row0 = base + off
                pltpu.make_async_copy(
                    t4_hbm.at[idx_v.at[pl.ds(0, CH)]], buf, gsem
                ).wait()

                def _drain_prev():
                    for t in range(FULLT):
                        pltpu.make_async_copy(
                            obuf.at[:, t],
                            out_hbm.at[pl.ds(row0, CH), pl.ds(t * 128, 128)],
                            ssem).wait()
                    pltpu.make_async_copy(
                        tail_v,
                        out_hbm.at[pl.ds(row0, CH),
                                   pl.ds(FULLT * 128, TAILC)],
                        ssem).wait()
                if b == 0:
                    pl.when(ci > 0)(_drain_prev)
                else:
                    _drain_prev()

                def _issue_next():
                    pltpu.async_copy(
                        t4_hbm.at[idx_v.at[pl.ds(off + CH, CH)]], obuf, gsem
                    )
                if b == 0:
                    _issue_next()
                else:
                    pl.when(ci + 2 < NCHUNK)(_issue_next)

                for g in range(G):
                    j0 = off + g * 16
                    ivec = idx_v[pl.ds(j0, 16)]
                    tvec = tgt_v[pl.ds(j0, 16)]
                    lsev = plsc.load_gather(lse_v, [ivec])
                    lrow = lanes + g * 16
                    tval = plsc.load_gather(
                        buf, [lrow, tvec // 128, tvec % 128])
                    acc_v[...] = acc_v[...] + (lsev - tval)

                @pl.loop(0, CH)
                def _row(j):
                    for g in range(FULLG):
                        tail_v[j, pl.ds(g * 16, 16)] = (
                            buf[j, FULLT, pl.ds(g * 16, 16)]
                        )
                    if REM:
                        colv = FULLG * 16 + lanes
                        v = buf[j, FULLT, pl.ds(FULLG * 16, 16)]
                        jv = jnp.zeros((16,), jnp.int32) + j
                        plsc.store_scatter(tail_v, [jv, colv],
                                           v, mask=colv < TAILC)

                for t in range(FULLT):
                    pltpu.async_copy(
                        buf.at[:, t],
                        out_hbm.at[pl.ds(row0, CH), pl.ds(t * 128, 128)],
                        ssem)
                pltpu.async_copy(
                    tail_v,
                    out_hbm.at[pl.ds(row0, CH), pl.ds(FULLT * 128, TAILC)],
                    ssem)

        for t in range(FULLT):
            pltpu.make_async_copy(
                rows_b.at[:, t],
                out_hbm.at[pl.ds(base, CH), pl.ds(t * 128, 128)],
                ssem).wait()
        pltpu.make_async_copy(
            tail_v,
            out_hbm.at[pl.ds(base, CH), pl.ds(FULLT * 128, TAILC)],
            ssem).wait()

        pltpu.sync_copy(acc_v, part_hbm.at[wid])

    return sc_kernel


def NCHUNK_EVEN(BPW, CH):
    return (BPW // CH) % 2 == 0


def _make_fin(N):
    def _fin_body(part_ref, loss_ref):
        loss_ref[...] = (jnp.sum(part_ref[...]) / N).reshape(1, 1)

    return pl.pallas_call(
        _fin_body,
        out_shape=jax.ShapeDtypeStruct((1, 1), jnp.float32),
    )


def kernel(idx, targets, table):
    B, T = idx.shape
    V, C = table.shape
    N = B * T
    NT = (C + 127) // 128
    idx_flat = idx.reshape(N).astype(jnp.int32)
    tgt_flat = targets.reshape(N).astype(jnp.int32)
    table = table.astype(jnp.float32)
    table4 = jnp.pad(table, ((0, 0), (0, NT * 128 - C))).reshape(V, NT, 128)

    lse = _compute_lse(table)
    logits, partials = _make_sc_gather(N, V, C)(
        table4, idx_flat, tgt_flat, lse
    )
    loss = _make_fin(N)(partials).reshape(())
    return (logits, loss)
